# baseline (speedup 1.0000x reference)
import functools

import jax
import jax.numpy as jnp
from jax import lax
from jax.experimental import pallas as pl
from jax.experimental.pallas import tpu as pltpu
from jax.experimental.pallas import tpu_sc as plsc

M = 2048
NN = 16
B = 2
J = 4
ENC = 5
PK = 128

NC = 2
NS = 16
NW = NC * NS
E = M * NN
EPW = E // NW
NCHUNK = 2
CH = EPW // NCHUNK
EPT = E // NS
DEDUP_ROUNDS = 4

_GDN = lax.GatherDimensionNumbers(offset_dims=(), collapsed_slice_dims=(0,),
                                  start_index_map=(0,))


def _shuf(vec, perm):
    return lax.gather(vec, perm.reshape(16, 1), _GDN, (1,),
                      mode=lax.GatherScatterMode.PROMISE_IN_BOUNDS)


def _xlane(vec, op):
    for k in (1, 2, 4, 8):
        perm = lax.bitwise_xor(lax.iota(jnp.int32, 16), k)
        vec = op(vec, _shuf(vec, perm))
    return vec


def _make_edges():
    mesh = plsc.VectorSubcoreMesh(core_axis_name="c", subcore_axis_name="s")
    out_type = (
        jax.ShapeDtypeStruct((E,), jnp.float32),
        jax.ShapeDtypeStruct((E,), jnp.float32),
        jax.ShapeDtypeStruct((E,), jnp.float32),
        jax.ShapeDtypeStruct((M * M + 8,), jnp.int32),
    )

    @functools.partial(
        pl.kernel, mesh=mesh, out_type=out_type,
        scratch_types=[
            pltpu.VMEM((EPT,), jnp.int32),
            pltpu.VMEM((EPT,), jnp.int32),
            pltpu.VMEM((EPT,), jnp.int32),
            pltpu.VMEM((EPT,), jnp.int32),
            pltpu.VMEM((EPT,), jnp.float32),
            pltpu.VMEM((EPT,), jnp.float32),
            pltpu.VMEM((EPT,), jnp.float32),
            pltpu.VMEM((EPT,), jnp.float32),
            pltpu.VMEM((EPT,), jnp.float32),
            pltpu.VMEM((EPT,), jnp.float32),
            pltpu.VMEM((EPT,), jnp.float32),
            pltpu.VMEM((EPT,), jnp.float32),
            pltpu.VMEM((80, 16), jnp.float32),
            pltpu.SemaphoreType.DMA,
        ],
    )
    def edges(lidx_hbm, eh0, eh1, eh2, eh3, eh4, tab_hbm,
              a0_hbm, a1_hbm, mask_hbm, w_hbm,
              idx_v, e_v, w_v, idx2_v, ec0, ec1, ec2, ec3, ec4,
              msk_v, a0_v, a1_v, tab_v, sem):
        ecs = (ec0, ec1, ec2, ec3, ec4)
        ehs = (eh0, eh1, eh2, eh3, eh4)
        c = lax.axis_index("c")
        s = lax.axis_index("s")
        base = s * EPT
        dummy = M * M

        @pl.when(c == 0)
        def _dedup():
            pltpu.sync_copy(lidx_hbm.at[pl.ds(base, EPT)], idx_v)

            def fill_e(k, _):
                e_v[pl.ds(k * 16, 16)] = lax.iota(jnp.int32, 16) + (base + k * 16)
                return 0
            lax.fori_loop(0, EPT // 16, fill_e, 0)
            pltpu.async_copy(e_v, w_hbm.at[idx_v], sem).wait()
            plsc.subcore_barrier()
            for _ in range(DEDUP_ROUNDS):
                pltpu.async_copy(w_hbm.at[idx_v], w_v, sem).wait()

                def claim(k, _):
                    sl = pl.ds(k * 16, 16)
                    idx2_v[sl] = jnp.where(w_v[sl] < e_v[sl], idx_v[sl], dummy)
                    return 0
                lax.fori_loop(0, EPT // 16, claim, 0)
                pltpu.async_copy(e_v, w_hbm.at[idx2_v], sem).wait()
                plsc.subcore_barrier()
            pltpu.async_copy(w_hbm.at[idx_v], w_v, sem).wait()

            def mk(k, _):
                sl = pl.ds(k * 16, 16)
                msk_v[sl] = jnp.where(w_v[sl] == e_v[sl], 1.0, 0.0)
                return 0
            lax.fori_loop(0, EPT // 16, mk, 0)
            pltpu.sync_copy(msk_v, mask_hbm.at[pl.ds(base, EPT)])

        @pl.when(c == 1)
        def _attention():
            for cc in range(ENC):
                pltpu.sync_copy(ehs[cc].at[pl.ds(base, EPT)], ecs[cc])
            pltpu.sync_copy(tab_hbm, tab_v)

            def grp(g, _):
                sl = pl.ds(g * 16, 16)
                emb = [ecs[cc][sl] for cc in range(ENC)]
                for l, av in ((0, a0_v), (1, a1_v)):
                    off = 40 * l
                    w = jnp.zeros((16,), jnp.float32)
                    for j in range(J):
                        t = jnp.zeros((16,), jnp.float32)
                        for cc in range(ENC):
                            u = emb[cc] - tab_v[off + j * 5 + cc]
                            t = t + u * tab_v[off + 20 + j * 5 + cc] * u
                        w = w + jnp.exp(-0.5 * t)
                    mx = _xlane(w, jnp.maximum)
                    ew = jnp.exp(w - mx)
                    av[sl] = ew / _xlane(ew, jnp.add)
                return 0
            lax.fori_loop(0, EPT // 16, grp, 0)
            pltpu.sync_copy(a0_v, a0_hbm.at[pl.ds(base, EPT)])
            pltpu.sync_copy(a1_v, a1_hbm.at[pl.ds(base, EPT)])

    return edges


_edges = _make_edges()


def _make_spmm(F):
    mesh = plsc.VectorSubcoreMesh(core_axis_name="c", subcore_axis_name="s")

    @functools.partial(
        pl.kernel, mesh=mesh,
        out_type=jax.ShapeDtypeStruct((NC, M, PK), jnp.float32),
        scratch_types=[
            pltpu.VMEM((EPW,), jnp.int32),
            pltpu.VMEM((CH,), jnp.int32),
            pltpu.VMEM((CH,), jnp.int32),
            pltpu.VMEM((CH,), jnp.int32),
            pltpu.VMEM((CH,), jnp.int32),
            pltpu.VMEM((EPW,), jnp.float32),
            pltpu.VMEM((EPW,), jnp.float32),
            pltpu.VMEM((CH, PK), jnp.float32),
            pltpu.VMEM_SHARED((M, PK), jnp.float32),
            pltpu.SemaphoreType.DMA,
        ],
    )
    def spmm(lidx_hbm, a_hbm, mask_hbm, hp_hbm, zeros_hbm, out_hbm,
             idx_v, n_v0, n_v1, m_v0, m_v1, a_v, mask_v, rows, acc, sem):
        n_vs = (n_v0, n_v1)
        m_vs = (m_v0, m_v1)
        c = lax.axis_index("c")
        s = lax.axis_index("s")
        wid = c * NS + s
        base = wid * EPW
        rpt = M // NS

        pltpu.sync_copy(lidx_hbm.at[pl.ds(base, EPW)], idx_v)
        pltpu.sync_copy(a_hbm.at[pl.ds(base, EPW)], a_v)
        pltpu.sync_copy(mask_hbm.at[pl.ds(base, EPW)], mask_v)

        def apply_mask(k, _):
            sl = pl.ds(k * 16, 16)
            a_v[sl] = a_v[sl] * mask_v[sl]
            return 0
        lax.fori_loop(0, EPW // 16, apply_mask, 0)

        for ch in range(NCHUNK):
            def decomp(k, _, ch=ch):
                vec = idx_v[pl.ds(ch * CH + k * 16, 16)]
                n_vs[ch][pl.ds(k * 16, 16)] = lax.shift_right_logical(vec, 11)
                m_vs[ch][pl.ds(k * 16, 16)] = lax.bitwise_and(vec, 2047)
                return 0
            lax.fori_loop(0, CH // 16, decomp, 0)

        pltpu.sync_copy(zeros_hbm, acc.at[pl.ds(s * rpt, rpt)])
        plsc.subcore_barrier()

        for ch in range(NCHUNK):
            pltpu.async_copy(hp_hbm.at[m_vs[ch]], rows, sem).wait()

            def scale(k, _):
                av = a_v[pl.ds(ch * CH + k * 16, 16)]
                for j in range(16):
                    avj = lax.gather(
                        av, jnp.full((16, 1), j, jnp.int32),
                        lax.GatherDimensionNumbers(offset_dims=(),
                                                   collapsed_slice_dims=(0,),
                                                   start_index_map=(0,)),
                        (1,), mode=lax.GatherScatterMode.PROMISE_IN_BOUNDS)
                    i = k * 16 + j
                    for b in range(B):
                        for fc in range(F // 16):
                            sl = pl.ds(b * 64 + fc * 16, 16)
                            rows[i, sl] = rows[i, sl] * avj
                return 0
            lax.fori_loop(0, CH // 16, scale, 0)

            pltpu.sync_copy(rows, acc.at[n_vs[ch]], add=True)

        plsc.subcore_barrier()
        pltpu.sync_copy(acc.at[pl.ds(s * rpt, rpt)],
                        out_hbm.at[c].at[pl.ds(s * rpt, rpt)])

    return spmm


_spmm16 = _make_spmm(16)
_spmm32 = _make_spmm(32)


def kernel(x, pseudo, L_idx, W_edge, b_edge, W1_0, b1_0, W2_0, b2_0, gamma_0, beta_0, sigma_0, mu_0, W1_1, b1_1, W2_1, b2_1, gamma_1, beta_1, sigma_1, mu_1, fc1_W, fc1_b, fc2_W, fc2_b):
    embed = pseudo.reshape(-1, 2) @ W_edge + b_edge
    rows = []
    for sigma, mu in ((sigma_0, mu_0), (sigma_1, mu_1)):
        rows += [mu[j, cc] for j in range(J) for cc in range(ENC)]
        rows += [sigma[j, cc] for j in range(J) for cc in range(ENC)]
    tab = jnp.broadcast_to(jnp.stack(rows)[:, None], (80, 16))
    a0, a1, mask, _ = _edges(L_idx, embed[:, 0], embed[:, 1], embed[:, 2],
                             embed[:, 3], embed[:, 4], tab)
    zeros = jnp.zeros((M // NS, PK), jnp.float32)

    layers = [(16, _spmm16, a0, W1_0, b1_0, W2_0, b2_0, gamma_0, beta_0),
              (32, _spmm32, a1, W1_1, b1_1, W2_1, b2_1, gamma_1, beta_1)]
    h = x
    for (F, spmm, a, W1, b1, W2, b2, gamma, beta) in layers:
        hp = jnp.zeros((M, PK), jnp.float32)
        hp = hp.at[:, 0:F].set(h[0]).at[:, 64:64 + F].set(h[1])
        partial = spmm(L_idx, a, mask, hp, zeros)
        psum = partial[0] + partial[1]
        Lx = jnp.stack([psum[:, 0:F], psum[:, 64:64 + F]])
        z = Lx @ W1 + b1 + h @ W2 + b2
        mean = jnp.mean(z, axis=(0, 1))
        var = jnp.var(z, axis=(0, 1))
        z = (z - mean) / jnp.sqrt(var + 1e-5) * gamma + beta
        h = jax.nn.relu(z)
    h = h.reshape(B, -1)
    h = jax.nn.relu(h @ fc1_W + fc1_b)
    return h @ fc2_W + fc2_b


# speedup vs baseline: 42.6018x; 42.6018x over previous
import functools

import jax
import jax.numpy as jnp
from jax import lax
from jax.experimental import pallas as pl
from jax.experimental.pallas import tpu as pltpu
from jax.experimental.pallas import tpu_sc as plsc

M = 2048
NN = 16
B = 2
J = 4
ENC = 5
PK = 128

NC = 2
NS = 16
NW = NC * NS
E = M * NN
EPW = E // NW
NCHUNK = 2
CH = EPW // NCHUNK
EPT = E // NS
DEDUP_ROUNDS = 3

_GDN = lax.GatherDimensionNumbers(offset_dims=(), collapsed_slice_dims=(0,),
                                  start_index_map=(0,))


def _shuf(vec, perm):
    return lax.gather(vec, perm.reshape(16, 1), _GDN, (1,),
                      mode=lax.GatherScatterMode.PROMISE_IN_BOUNDS)


def _xlane(vec, op):
    for k in (1, 2, 4, 8):
        perm = lax.bitwise_xor(lax.iota(jnp.int32, 16), k)
        vec = op(vec, _shuf(vec, perm))
    return vec


def _make_edges():
    mesh = plsc.VectorSubcoreMesh(core_axis_name="c", subcore_axis_name="s")
    out_type = (
        jax.ShapeDtypeStruct((E,), jnp.float32),
        jax.ShapeDtypeStruct((E,), jnp.float32),
        jax.ShapeDtypeStruct((E,), jnp.float32),
        jax.ShapeDtypeStruct((E,), jnp.float32),
    )
    QW = 1 << 20

    @functools.partial(
        pl.kernel, mesh=mesh, out_type=out_type,
        scratch_types=[
            pltpu.VMEM((EPT,), jnp.int32),
            pltpu.VMEM((EPT,), jnp.int32),
            pltpu.VMEM((EPT,), jnp.int32),
            pltpu.VMEM((EPT,), jnp.int32),
            pltpu.VMEM((EPT,), jnp.int32),
            pltpu.VMEM((EPT,), jnp.float32),
            pltpu.VMEM((EPW,), jnp.float32),
            pltpu.VMEM((EPW,), jnp.float32),
            pltpu.VMEM((EPW,), jnp.float32),
            pltpu.VMEM((EPW,), jnp.float32),
            pltpu.VMEM((EPW,), jnp.float32),
            pltpu.VMEM((EPW,), jnp.float32),
            pltpu.VMEM((EPW,), jnp.float32),
            pltpu.VMEM((80, 16), jnp.float32),
            pltpu.VMEM_SHARED((QW + 8,), jnp.int32),
        ],
    )
    def edges(lidx_hbm, eh0, eh1, eh2, eh3, eh4, tab_hbm,
              a0_hbm, a1_hbm, mask0_hbm, mask1_hbm,
              idx_v, e_v, w_v, idxq_v, idx2_v, msk_v,
              ec0, ec1, ec2, ec3, ec4, a0_v, a1_v, tab_v, w_sp):
        ecs = (ec0, ec1, ec2, ec3, ec4)
        ehs = (eh0, eh1, eh2, eh3, eh4)
        c = lax.axis_index("c")
        s = lax.axis_index("s")
        wid = c * NS + s
        base_a = wid * EPW
        base_d = s * EPT

        for cc in range(ENC):
            pltpu.sync_copy(ehs[cc].at[pl.ds(base_a, EPW)], ecs[cc])
        pltpu.sync_copy(tab_hbm, tab_v)

        def grp(g, _):
            sl = pl.ds(g * 16, 16)
            emb = [ecs[cc][sl] for cc in range(ENC)]
            for l, av in ((0, a0_v), (1, a1_v)):
                off = 40 * l
                w = jnp.zeros((16,), jnp.float32)
                for j in range(J):
                    t = jnp.zeros((16,), jnp.float32)
                    for cc in range(ENC):
                        u = emb[cc] - tab_v[off + j * 5 + cc]
                        t = t + u * tab_v[off + 20 + j * 5 + cc] * u
                    w = w + jnp.exp(-0.5 * t)
                mx = _xlane(w, jnp.maximum)
                ew = jnp.exp(w - mx)
                av[sl] = ew / _xlane(ew, jnp.add)
            return 0
        lax.fori_loop(0, EPW // 16, grp, 0)
        pltpu.sync_copy(a0_v, a0_hbm.at[pl.ds(base_a, EPW)])
        pltpu.sync_copy(a1_v, a1_hbm.at[pl.ds(base_a, EPW)])

        pltpu.sync_copy(lidx_hbm.at[pl.ds(base_d, EPT)], idx_v)

        def fill_e(k, _):
            e_v[pl.ds(k * 16, 16)] = lax.iota(jnp.int32, 16) + (base_d + k * 16)
            return 0
        lax.fori_loop(0, EPT // 16, fill_e, 0)

        for p in range(2):
            qbase = (c * 2 + p) * QW

            def quarterize(k, _):
                sl = pl.ds(k * 16, 16)
                loc = idx_v[sl] - qbase
                inq = jnp.logical_and(loc >= 0, loc < QW)
                idxq_v[sl] = jnp.where(inq, loc, QW)
                return 0
            lax.fori_loop(0, EPT // 16, quarterize, 0)

            pltpu.sync_copy(e_v, w_sp.at[idxq_v])
            plsc.subcore_barrier()
            for _ in range(DEDUP_ROUNDS):
                pltpu.sync_copy(w_sp.at[idxq_v], w_v)

                def claim(k, _):
                    sl = pl.ds(k * 16, 16)
                    good = jnp.logical_and(idxq_v[sl] < QW, w_v[sl] < e_v[sl])
                    idx2_v[sl] = jnp.where(good, idxq_v[sl], QW)
                    return 0
                lax.fori_loop(0, EPT // 16, claim, 0)
                pltpu.sync_copy(e_v, w_sp.at[idx2_v])
                plsc.subcore_barrier()
            pltpu.sync_copy(w_sp.at[idxq_v], w_v)

            def mk(k, _, p=p):
                sl = pl.ds(k * 16, 16)
                win = jnp.logical_and(idxq_v[sl] < QW, w_v[sl] == e_v[sl])
                add = jnp.where(win, 1.0, 0.0)
                msk_v[sl] = add if p == 0 else msk_v[sl] + add
                return 0
            lax.fori_loop(0, EPT // 16, mk, 0)
            plsc.subcore_barrier()

        @pl.when(c == 0)
        def _w0():
            pltpu.sync_copy(msk_v, mask0_hbm.at[pl.ds(base_d, EPT)])

        @pl.when(c == 1)
        def _w1():
            pltpu.sync_copy(msk_v, mask1_hbm.at[pl.ds(base_d, EPT)])

    return edges


_edges = _make_edges()


def _make_spmm(F):
    mesh = plsc.VectorSubcoreMesh(core_axis_name="c", subcore_axis_name="s")

    @functools.partial(
        pl.kernel, mesh=mesh,
        out_type=jax.ShapeDtypeStruct((NC, M, PK), jnp.float32),
        scratch_types=[
            pltpu.VMEM((EPW,), jnp.int32),
            pltpu.VMEM((CH,), jnp.int32),
            pltpu.VMEM((CH,), jnp.int32),
            pltpu.VMEM((CH,), jnp.int32),
            pltpu.VMEM((CH,), jnp.int32),
            pltpu.VMEM((EPW,), jnp.float32),
            pltpu.VMEM((EPW,), jnp.float32),
            pltpu.VMEM((EPW,), jnp.float32),
            pltpu.VMEM((CH, PK), jnp.float32),
            pltpu.VMEM_SHARED((M, PK), jnp.float32),
            pltpu.SemaphoreType.DMA,
        ],
    )
    def spmm(lidx_hbm, a_hbm, mask0_hbm, mask1_hbm, hp_hbm, zeros_hbm, out_hbm,
             idx_v, n_v0, n_v1, m_v0, m_v1, a_v, mask_v, mask2_v, rows, acc, sem):
        n_vs = (n_v0, n_v1)
        m_vs = (m_v0, m_v1)
        c = lax.axis_index("c")
        s = lax.axis_index("s")
        wid = c * NS + s
        base = wid * EPW
        rpt = M // NS

        pltpu.sync_copy(lidx_hbm.at[pl.ds(base, EPW)], idx_v)
        pltpu.sync_copy(a_hbm.at[pl.ds(base, EPW)], a_v)
        pltpu.sync_copy(mask0_hbm.at[pl.ds(base, EPW)], mask_v)
        pltpu.sync_copy(mask1_hbm.at[pl.ds(base, EPW)], mask2_v)

        def apply_mask(k, _):
            sl = pl.ds(k * 16, 16)
            a_v[sl] = a_v[sl] * (mask_v[sl] + mask2_v[sl])
            return 0
        lax.fori_loop(0, EPW // 16, apply_mask, 0)

        for ch in range(NCHUNK):
            def decomp(k, _, ch=ch):
                vec = idx_v[pl.ds(ch * CH + k * 16, 16)]
                n_vs[ch][pl.ds(k * 16, 16)] = lax.shift_right_logical(vec, 11)
                m_vs[ch][pl.ds(k * 16, 16)] = lax.bitwise_and(vec, 2047)
                return 0
            lax.fori_loop(0, CH // 16, decomp, 0)

        pltpu.sync_copy(zeros_hbm, acc.at[pl.ds(s * rpt, rpt)])
        plsc.subcore_barrier()

        for ch in range(NCHUNK):
            pltpu.async_copy(hp_hbm.at[m_vs[ch]], rows, sem).wait()

            def scale(k, _):
                av = a_v[pl.ds(ch * CH + k * 16, 16)]
                for j in range(16):
                    avj = lax.gather(
                        av, jnp.full((16, 1), j, jnp.int32),
                        lax.GatherDimensionNumbers(offset_dims=(),
                                                   collapsed_slice_dims=(0,),
                                                   start_index_map=(0,)),
                        (1,), mode=lax.GatherScatterMode.PROMISE_IN_BOUNDS)
                    i = k * 16 + j
                    for b in range(B):
                        for fc in range(F // 16):
                            sl = pl.ds(b * 64 + fc * 16, 16)
                            rows[i, sl] = rows[i, sl] * avj
                return 0
            lax.fori_loop(0, CH // 16, scale, 0)

            pltpu.sync_copy(rows, acc.at[n_vs[ch]], add=True)

        plsc.subcore_barrier()
        pltpu.sync_copy(acc.at[pl.ds(s * rpt, rpt)],
                        out_hbm.at[c].at[pl.ds(s * rpt, rpt)])

    return spmm


_spmm16 = _make_spmm(16)
_spmm32 = _make_spmm(32)


def kernel(x, pseudo, L_idx, W_edge, b_edge, W1_0, b1_0, W2_0, b2_0, gamma_0, beta_0, sigma_0, mu_0, W1_1, b1_1, W2_1, b2_1, gamma_1, beta_1, sigma_1, mu_1, fc1_W, fc1_b, fc2_W, fc2_b):
    embed = pseudo.reshape(-1, 2) @ W_edge + b_edge
    rows = []
    for sigma, mu in ((sigma_0, mu_0), (sigma_1, mu_1)):
        rows += [mu[j, cc] for j in range(J) for cc in range(ENC)]
        rows += [sigma[j, cc] for j in range(J) for cc in range(ENC)]
    tab = jnp.broadcast_to(jnp.stack(rows)[:, None], (80, 16))
    a0, a1, mask0, mask1 = _edges(L_idx, embed[:, 0], embed[:, 1], embed[:, 2],
                                  embed[:, 3], embed[:, 4], tab)
    zeros = jnp.zeros((M // NS, PK), jnp.float32)

    layers = [(16, _spmm16, a0, W1_0, b1_0, W2_0, b2_0, gamma_0, beta_0),
              (32, _spmm32, a1, W1_1, b1_1, W2_1, b2_1, gamma_1, beta_1)]
    h = x
    for (F, spmm, a, W1, b1, W2, b2, gamma, beta) in layers:
        hp = jnp.zeros((M, PK), jnp.float32)
        hp = hp.at[:, 0:F].set(h[0]).at[:, 64:64 + F].set(h[1])
        partial = spmm(L_idx, a, mask0, mask1, hp, zeros)
        psum = partial[0] + partial[1]
        Lx = jnp.stack([psum[:, 0:F], psum[:, 64:64 + F]])
        z = Lx @ W1 + b1 + h @ W2 + b2
        mean = jnp.mean(z, axis=(0, 1))
        var = jnp.var(z, axis=(0, 1))
        z = (z - mean) / jnp.sqrt(var + 1e-5) * gamma + beta
        h = jax.nn.relu(z)
    h = h.reshape(B, -1)
    h = jax.nn.relu(h @ fc1_W + fc1_b)
    return h @ fc2_W + fc2_b


# speedup vs baseline: 79.5050x; 1.8662x over previous
import functools

import jax
import jax.numpy as jnp
from jax import lax
from jax.experimental import pallas as pl
from jax.experimental.pallas import tpu as pltpu
from jax.experimental.pallas import tpu_sc as plsc

M = 2048
NN = 16
B = 2
J = 4
ENC = 5
PK = 128

NC = 2
NS = 16
NW = NC * NS
E = M * NN
EPW = E // NW
NCHUNK = 2
CH = EPW // NCHUNK
EPT = E // NS
DEDUP_ROUNDS = 3

_GDN = lax.GatherDimensionNumbers(offset_dims=(), collapsed_slice_dims=(0,),
                                  start_index_map=(0,))


def _shuf(vec, perm):
    return lax.gather(vec, perm.reshape(16, 1), _GDN, (1,),
                      mode=lax.GatherScatterMode.PROMISE_IN_BOUNDS)


def _xlane(vec, op):
    for k in (1, 2, 4, 8):
        perm = lax.bitwise_xor(lax.iota(jnp.int32, 16), k)
        vec = op(vec, _shuf(vec, perm))
    return vec


def _make_edges():
    mesh = plsc.VectorSubcoreMesh(core_axis_name="c", subcore_axis_name="s")
    out_type = (
        jax.ShapeDtypeStruct((E,), jnp.float32),
        jax.ShapeDtypeStruct((E,), jnp.float32),
    )

    @functools.partial(
        pl.kernel, mesh=mesh, out_type=out_type,
        scratch_types=[
            pltpu.VMEM((EPW,), jnp.float32),
            pltpu.VMEM((EPW,), jnp.float32),
            pltpu.VMEM((EPW,), jnp.float32),
            pltpu.VMEM((EPW,), jnp.float32),
            pltpu.VMEM((EPW,), jnp.float32),
            pltpu.VMEM((EPW,), jnp.float32),
            pltpu.VMEM((EPW,), jnp.float32),
            pltpu.VMEM((80, 16), jnp.float32),
        ],
    )
    def edges(eh0, eh1, eh2, eh3, eh4, tab_hbm,
              a0_hbm, a1_hbm,
              ec0, ec1, ec2, ec3, ec4, a0_v, a1_v, tab_v):
        ecs = (ec0, ec1, ec2, ec3, ec4)
        ehs = (eh0, eh1, eh2, eh3, eh4)
        c = lax.axis_index("c")
        s = lax.axis_index("s")
        wid = c * NS + s
        base_a = wid * EPW

        for cc in range(ENC):
            pltpu.sync_copy(ehs[cc].at[pl.ds(base_a, EPW)], ecs[cc])
        pltpu.sync_copy(tab_hbm, tab_v)

        def grp(g, _):
            sl = pl.ds(g * 16, 16)
            emb = [ecs[cc][sl] for cc in range(ENC)]
            for l, av in ((0, a0_v), (1, a1_v)):
                off = 40 * l
                w = jnp.zeros((16,), jnp.float32)
                for j in range(J):
                    t = jnp.zeros((16,), jnp.float32)
                    for cc in range(ENC):
                        u = emb[cc] - tab_v[off + j * 5 + cc]
                        t = t + u * tab_v[off + 20 + j * 5 + cc] * u
                    w = w + jnp.exp(-0.5 * t)
                mx = _xlane(w, jnp.maximum)
                ew = jnp.exp(w - mx)
                av[sl] = ew / _xlane(ew, jnp.add)
            return 0
        lax.fori_loop(0, EPW // 16, grp, 0)
        pltpu.sync_copy(a0_v, a0_hbm.at[pl.ds(base_a, EPW)])
        pltpu.sync_copy(a1_v, a1_hbm.at[pl.ds(base_a, EPW)])

    return edges


_edges = _make_edges()


def _make_spmm(F):
    mesh = plsc.VectorSubcoreMesh(core_axis_name="c", subcore_axis_name="s")

    @functools.partial(
        pl.kernel, mesh=mesh,
        out_type=jax.ShapeDtypeStruct((NC, M, PK), jnp.float32),
        scratch_types=[
            pltpu.VMEM((EPW,), jnp.int32),
            pltpu.VMEM((CH,), jnp.int32),
            pltpu.VMEM((CH,), jnp.int32),
            pltpu.VMEM((CH,), jnp.int32),
            pltpu.VMEM((CH,), jnp.int32),
            pltpu.VMEM((EPW,), jnp.float32),
            pltpu.VMEM((EPW,), jnp.float32),
            pltpu.VMEM((EPW,), jnp.float32),
            pltpu.VMEM((CH, PK), jnp.float32),
            pltpu.VMEM_SHARED((M, PK), jnp.float32),
            pltpu.SemaphoreType.DMA,
        ],
    )
    def spmm(lidx_hbm, a_hbm, mask0_hbm, mask1_hbm, hp_hbm, zeros_hbm, out_hbm,
             idx_v, n_v0, n_v1, m_v0, m_v1, a_v, mask_v, mask2_v, rows, acc, sem):
        n_vs = (n_v0, n_v1)
        m_vs = (m_v0, m_v1)
        c = lax.axis_index("c")
        s = lax.axis_index("s")
        wid = c * NS + s
        base = wid * EPW
        rpt = M // NS

        pltpu.sync_copy(lidx_hbm.at[pl.ds(base, EPW)], idx_v)
        pltpu.sync_copy(a_hbm.at[pl.ds(base, EPW)], a_v)
        pltpu.sync_copy(mask0_hbm.at[pl.ds(base, EPW)], mask_v)
        pltpu.sync_copy(mask1_hbm.at[pl.ds(base, EPW)], mask2_v)

        def apply_mask(k, _):
            sl = pl.ds(k * 16, 16)
            a_v[sl] = a_v[sl] * (mask_v[sl] + mask2_v[sl])
            return 0
        lax.fori_loop(0, EPW // 16, apply_mask, 0)

        for ch in range(NCHUNK):
            def decomp(k, _, ch=ch):
                vec = idx_v[pl.ds(ch * CH + k * 16, 16)]
                n_vs[ch][pl.ds(k * 16, 16)] = lax.shift_right_logical(vec, 11)
                m_vs[ch][pl.ds(k * 16, 16)] = lax.bitwise_and(vec, 2047)
                return 0
            lax.fori_loop(0, CH // 16, decomp, 0)

        pltpu.sync_copy(zeros_hbm, acc.at[pl.ds(s * rpt, rpt)])
        plsc.subcore_barrier()

        for ch in range(NCHUNK):
            pltpu.async_copy(hp_hbm.at[m_vs[ch]], rows, sem).wait()

            def scale(k, _):
                av = a_v[pl.ds(ch * CH + k * 16, 16)]
                for j in range(16):
                    avj = lax.gather(
                        av, jnp.full((16, 1), j, jnp.int32),
                        lax.GatherDimensionNumbers(offset_dims=(),
                                                   collapsed_slice_dims=(0,),
                                                   start_index_map=(0,)),
                        (1,), mode=lax.GatherScatterMode.PROMISE_IN_BOUNDS)
                    i = k * 16 + j
                    for b in range(B):
                        for fc in range(F // 16):
                            sl = pl.ds(b * 64 + fc * 16, 16)
                            rows[i, sl] = rows[i, sl] * avj
                return 0
            lax.fori_loop(0, CH // 16, scale, 0)

            pltpu.sync_copy(rows, acc.at[n_vs[ch]], add=True)

        plsc.subcore_barrier()
        pltpu.sync_copy(acc.at[pl.ds(s * rpt, rpt)],
                        out_hbm.at[c].at[pl.ds(s * rpt, rpt)])

    return spmm


_spmm16 = _make_spmm(16)
_spmm32 = _make_spmm(32)


def kernel(x, pseudo, L_idx, W_edge, b_edge, W1_0, b1_0, W2_0, b2_0, gamma_0, beta_0, sigma_0, mu_0, W1_1, b1_1, W2_1, b2_1, gamma_1, beta_1, sigma_1, mu_1, fc1_W, fc1_b, fc2_W, fc2_b):
    embed = pseudo.reshape(-1, 2) @ W_edge + b_edge
    rows = []
    for sigma, mu in ((sigma_0, mu_0), (sigma_1, mu_1)):
        rows += [mu[j, cc] for j in range(J) for cc in range(ENC)]
        rows += [sigma[j, cc] for j in range(J) for cc in range(ENC)]
    tab = jnp.broadcast_to(jnp.stack(rows)[:, None], (80, 16))
    a0, a1 = _edges(embed[:, 0], embed[:, 1], embed[:, 2],
                    embed[:, 3], embed[:, 4], tab)
    eid = jnp.arange(E, dtype=jnp.int32)
    wbuf = jnp.zeros((M * M,), dtype=jnp.int32).at[L_idx].max(eid + 1)
    mask0 = (wbuf[L_idx] == eid + 1).astype(jnp.float32)
    mask1 = jnp.zeros((E,), jnp.float32)
    zeros = jnp.zeros((M // NS, PK), jnp.float32)

    layers = [(16, _spmm16, a0, W1_0, b1_0, W2_0, b2_0, gamma_0, beta_0),
              (32, _spmm32, a1, W1_1, b1_1, W2_1, b2_1, gamma_1, beta_1)]
    h = x
    for (F, spmm, a, W1, b1, W2, b2, gamma, beta) in layers:
        hp = jnp.zeros((M, PK), jnp.float32)
        hp = hp.at[:, 0:F].set(h[0]).at[:, 64:64 + F].set(h[1])
        partial = spmm(L_idx, a, mask0, mask1, hp, zeros)
        psum = partial[0] + partial[1]
        Lx = jnp.stack([psum[:, 0:F], psum[:, 64:64 + F]])
        z = Lx @ W1 + b1 + h @ W2 + b2
        mean = jnp.mean(z, axis=(0, 1))
        var = jnp.var(z, axis=(0, 1))
        z = (z - mean) / jnp.sqrt(var + 1e-5) * gamma + beta
        h = jax.nn.relu(z)
    h = h.reshape(B, -1)
    h = jax.nn.relu(h @ fc1_W + fc1_b)
    return h @ fc2_W + fc2_b
